# baseline (speedup 1.0000x reference)
import functools

import jax
import jax.numpy as jnp
from jax import lax
from jax.experimental import pallas as pl
from jax.experimental.pallas import tpu as pltpu
from jax.experimental.pallas import tpu_sc as plsc

N_NODES = 10000
N_EDGES = 320000
D = 128
EPS = 1e-5

NC = 2
NS = 16
NW = NC * NS
GROUP = 128
E_PAD = 327680
GP_W = (E_PAD // NW) // GROUP
N_GROUPS = E_PAD // GROUP
ROWS_PAD = 10112
RPT = ROWS_PAD // NS
NFULL = RPT // GROUP
TAIL = RPT - NFULL * GROUP


def _sc_aggregate(src2d, dst2d, x, zrow, onerow):
    mesh = plsc.VectorSubcoreMesh(core_axis_name="c", subcore_axis_name="s",
                                  num_cores=NC)

    @functools.partial(
        pl.kernel,
        mesh=mesh,
        out_type=[
            jax.ShapeDtypeStruct((NC, ROWS_PAD, D), jnp.float32),
            jax.ShapeDtypeStruct((NC, ROWS_PAD, D), jnp.float32),
        ],
        scratch_types=[
            pltpu.VMEM((4, GROUP), jnp.int32),
            pltpu.VMEM((4, GROUP), jnp.int32),
            pltpu.VMEM((GROUP, D), jnp.float32),
            pltpu.VMEM((GROUP, D), jnp.float32),
            pltpu.VMEM_SHARED((ROWS_PAD, D), jnp.float32),
            pltpu.SemaphoreType.DMA,
            pltpu.SemaphoreType.DMA,
            pltpu.SemaphoreType.DMA,
            pltpu.SemaphoreType.DMA,
        ],
    )
    def body(src_hbm, dst_hbm, x_hbm, zrow_hbm, onerow_hbm,
             agg_out, deg_out,
             srcv, dstv, rows0, rows1, acc_sh,
             gsem0, gsem1, ssem0, ssem1):
        c = lax.axis_index("c")
        s = lax.axis_index("s")
        wid = s * NC + c
        rbase = s * RPT
        gbase = wid * GP_W
        bufs = (rows0, rows1)
        gsems = (gsem0, gsem1)
        ssems = (ssem0, ssem1)

        def acc_chunk(t):
            n = TAIL if t == NFULL else GROUP
            return acc_sh.at[pl.ds(rbase + t * GROUP, n)]

        def zero_acc():
            pltpu.sync_copy(zrow_hbm, rows0)
            for t in range(NFULL):
                pltpu.async_copy(rows0, acc_chunk(t), ssem0)
            pltpu.async_copy(rows0.at[pl.ds(0, TAIL)], acc_chunk(NFULL), ssem1)
            for t in range(NFULL):
                pltpu.make_async_copy(rows0, acc_chunk(t), ssem0).wait()
            pltpu.make_async_copy(rows0.at[pl.ds(0, TAIL)], acc_chunk(NFULL),
                                  ssem1).wait()

        def copy_out(out_ref):
            def out_chunk(t):
                n = TAIL if t == NFULL else GROUP
                return out_ref.at[c, pl.ds(rbase + t * GROUP, n)]

            def buf_chunk(p, t):
                n = TAIL if t == NFULL else GROUP
                return bufs[p].at[pl.ds(0, n)]

            for t in range(NFULL + 1):
                p = t % 2
                if t >= 2:
                    pltpu.make_async_copy(buf_chunk(p, t - 2), out_chunk(t - 2),
                                          ssems[p]).wait()
                pltpu.async_copy(acc_chunk(t), buf_chunk(p, t), gsems[p]).wait()
                pltpu.async_copy(buf_chunk(p, t), out_chunk(t), ssems[p])
            for t in (NFULL - 1, NFULL):
                p = t % 2
                pltpu.make_async_copy(buf_chunk(p, t), out_chunk(t),
                                      ssems[p]).wait()

        zero_acc()
        plsc.subcore_barrier()

        def outer1(t, carry):
            pltpu.sync_copy(src_hbm.at[pl.ds(gbase + t * 4, 4)], srcv)
            pltpu.sync_copy(dst_hbm.at[pl.ds(gbase + t * 4, 4)], dstv)
            pltpu.async_copy(x_hbm.at[srcv.at[0]], rows0, gsem0)
            pltpu.async_copy(x_hbm.at[srcv.at[1]], rows1, gsem1)
            for j in range(4):
                p = j % 2
                pltpu.make_async_copy(x_hbm.at[srcv.at[j]], bufs[p],
                                      gsems[p]).wait()
                pltpu.async_copy(bufs[p], acc_sh.at[dstv.at[j]], ssems[p],
                                 add=True)
                if j + 2 < 4:
                    pltpu.make_async_copy(bufs[p], acc_sh.at[dstv.at[j]],
                                          ssems[p]).wait()
                    pltpu.async_copy(x_hbm.at[srcv.at[j + 2]], bufs[p],
                                     gsems[p])
            for j in (2, 3):
                p = j % 2
                pltpu.make_async_copy(bufs[p], acc_sh.at[dstv.at[j]],
                                      ssems[p]).wait()
            return carry

        lax.fori_loop(0, GP_W // 4, outer1, 0)
        plsc.subcore_barrier()
        copy_out(agg_out)
        plsc.subcore_barrier()

        zero_acc()
        pltpu.sync_copy(onerow_hbm, rows0)
        plsc.subcore_barrier()

        def outer2(t, carry):
            pltpu.sync_copy(dst_hbm.at[pl.ds(gbase + t * 4, 4)], dstv)
            for j in range(4):
                pltpu.async_copy(rows0, acc_sh.at[dstv.at[j]], ssem0, add=True)
            for j in range(4):
                pltpu.make_async_copy(rows0, acc_sh.at[dstv.at[j]],
                                      ssem0).wait()
            return carry

        lax.fori_loop(0, GP_W // 4, outer2, 0)
        plsc.subcore_barrier()
        copy_out(deg_out)

    return body(src2d, dst2d, x, zrow, onerow)


def _tc_body(p_ref, deg_ref, x_ref, wl_ref, wr_ref, bl_ref, g_ref, b_ref,
             o_ref):
    agg = p_ref[0] + p_ref[1]
    deg = deg_ref[0] + deg_ref[1]
    inv = 1.0 / jnp.maximum(deg[:, 0:1], 1.0)
    agg = agg * inv
    x = x_ref[...]
    f = (jnp.dot(agg, wl_ref[...], preferred_element_type=jnp.float32)
         + jnp.dot(x, wr_ref[...], preferred_element_type=jnp.float32)
         + bl_ref[...])
    f = 0.5 * f * (1.0 + lax.erf(f * 0.7071067811865476))
    mu = jnp.mean(f, axis=-1, keepdims=True)
    var = jnp.mean((f - mu) ** 2, axis=-1, keepdims=True)
    f = (f - mu) * lax.rsqrt(var + EPS) * g_ref[...] + b_ref[...]
    o_ref[...] = f + x


def _tc_finish(agg_p, deg_p, x, W_l, W_r, b_l, gamma, beta):
    R = 1000
    grid = (N_NODES // R,)
    return pl.pallas_call(
        _tc_body,
        grid=grid,
        in_specs=[
            pl.BlockSpec((NC, R, D), lambda i: (0, i, 0)),
            pl.BlockSpec((NC, R, D), lambda i: (0, i, 0)),
            pl.BlockSpec((R, D), lambda i: (i, 0)),
            pl.BlockSpec((D, D), lambda i: (0, 0)),
            pl.BlockSpec((D, D), lambda i: (0, 0)),
            pl.BlockSpec((1, D), lambda i: (0, 0)),
            pl.BlockSpec((1, D), lambda i: (0, 0)),
            pl.BlockSpec((1, D), lambda i: (0, 0)),
        ],
        out_specs=pl.BlockSpec((R, D), lambda i: (i, 0)),
        out_shape=jax.ShapeDtypeStruct((N_NODES, D), jnp.float32),
    )(agg_p, deg_p, x, W_l, W_r, b_l.reshape(1, D), gamma.reshape(1, D),
      beta.reshape(1, D))


def kernel(x, edge_index, batch, W_l, b_l, W_r, gamma, beta):
    src = edge_index[0].astype(jnp.int32)
    dst = edge_index[1].astype(jnp.int32)
    npad = E_PAD - N_EDGES
    src2d = jnp.concatenate(
        [src, jnp.zeros((npad,), jnp.int32)]).reshape(N_GROUPS, GROUP)
    dst2d = jnp.concatenate(
        [dst, jnp.full((npad,), N_NODES, jnp.int32)]).reshape(N_GROUPS, GROUP)
    zrow = jnp.zeros((GROUP, D), jnp.float32)
    onerow = jnp.ones((GROUP, D), jnp.float32)
    agg_p, deg_p = _sc_aggregate(src2d, dst2d, x, zrow, onerow)
    return _tc_finish(agg_p, deg_p, x, W_l, W_r, b_l, gamma, beta)


# speedup vs baseline: 1.1366x; 1.1366x over previous
import functools

import jax
import jax.numpy as jnp
from jax import lax
from jax.experimental import pallas as pl
from jax.experimental.pallas import tpu as pltpu
from jax.experimental.pallas import tpu_sc as plsc

N_NODES = 10000
N_EDGES = 320000
D = 128
EPS = 1e-5

NC = 2
NS = 16
NW = NC * NS
LANES = 16
GROUP = 64
E_PAD = 327680
GP_W = (E_PAD // NW) // GROUP
N_GROUPS = E_PAD // GROUP
ROWS_PAD = 10112
RPT = ROWS_PAD // NS
NFULL = RPT // GROUP
TAIL = RPT - NFULL * GROUP


def _sc_aggregate(src2d, dst2d, x, zrow):
    mesh = plsc.VectorSubcoreMesh(core_axis_name="c", subcore_axis_name="s",
                                  num_cores=NC)

    @functools.partial(
        pl.kernel,
        mesh=mesh,
        compiler_params=pltpu.CompilerParams(needs_layout_passes=False),
        out_type=[
            jax.ShapeDtypeStruct((NC, ROWS_PAD, D), jnp.float32),
            jax.ShapeDtypeStruct((NC, NS, ROWS_PAD), jnp.float32),
        ],
        scratch_types=[
            pltpu.VMEM((8, GROUP), jnp.int32),
            pltpu.VMEM((8, GROUP), jnp.int32),
            pltpu.VMEM((GROUP, D), jnp.float32),
            pltpu.VMEM((GROUP, D), jnp.float32),
            pltpu.VMEM((ROWS_PAD,), jnp.float32),
            pltpu.VMEM_SHARED((ROWS_PAD, D), jnp.float32),
            pltpu.SemaphoreType.DMA,
            pltpu.SemaphoreType.DMA,
            pltpu.SemaphoreType.DMA,
            pltpu.SemaphoreType.DMA,
        ],
    )
    def body(src_hbm, dst_hbm, x_hbm, zrow_hbm,
             agg_out, hist_out,
             srcv, dstv, rows0, rows1, hist, acc_sh,
             gsem0, gsem1, ssem0, ssem1):
        c = lax.axis_index("c")
        s = lax.axis_index("s")
        wid = s * NC + c
        rbase = s * RPT
        gbase = wid * GP_W
        bufs = (rows0, rows1)
        gsems = (gsem0, gsem1)
        ssems = (ssem0, ssem1)

        def acc_chunk(t):
            n = TAIL if t == NFULL else GROUP
            return acc_sh.at[pl.ds(rbase + t * GROUP, n)]

        zvec = jnp.zeros((LANES,), jnp.float32)

        def zh(i, carry):
            hist[pl.ds(i * LANES, LANES)] = zvec
            return carry

        lax.fori_loop(0, ROWS_PAD // LANES, zh, 0)

        pltpu.sync_copy(zrow_hbm, rows0)
        for t in range(NFULL):
            pltpu.async_copy(rows0, acc_chunk(t), ssem0)
        pltpu.async_copy(rows0.at[pl.ds(0, TAIL)], acc_chunk(NFULL), ssem1)
        for t in range(NFULL):
            pltpu.make_async_copy(rows0, acc_chunk(t), ssem0).wait()
        pltpu.make_async_copy(rows0.at[pl.ds(0, TAIL)], acc_chunk(NFULL),
                              ssem1).wait()
        plsc.subcore_barrier()

        def outer(t, carry):
            pltpu.sync_copy(src_hbm.at[pl.ds(gbase + t * 8, 8)], srcv)
            pltpu.sync_copy(dst_hbm.at[pl.ds(gbase + t * 8, 8)], dstv)
            pltpu.async_copy(x_hbm.at[srcv.at[0]], rows0, gsem0)
            pltpu.async_copy(x_hbm.at[srcv.at[1]], rows1, gsem1)
            for j in range(8):
                for k in range(GROUP // LANES):
                    d16 = dstv[j, pl.ds(k * LANES, LANES)]
                    cnt, last = plsc.scan_count(d16)
                    plsc.addupdate_scatter(hist, [d16],
                                           cnt.astype(jnp.float32), mask=last)
            for j in range(8):
                p = j % 2
                pltpu.make_async_copy(x_hbm.at[srcv.at[j]], bufs[p],
                                      gsems[p]).wait()
                pltpu.async_copy(bufs[p], acc_sh.at[dstv.at[j]], ssems[p],
                                 add=True)
                if j + 2 < 8:
                    pltpu.make_async_copy(bufs[p], acc_sh.at[dstv.at[j]],
                                          ssems[p]).wait()
                    pltpu.async_copy(x_hbm.at[srcv.at[j + 2]], bufs[p],
                                     gsems[p])
            for j in (6, 7):
                p = j % 2
                pltpu.make_async_copy(bufs[p], acc_sh.at[dstv.at[j]],
                                      ssems[p]).wait()
            return carry

        lax.fori_loop(0, GP_W // 8, outer, 0)
        pltpu.sync_copy(hist, hist_out.at[c, s])
        plsc.subcore_barrier()

        def out_chunk(t):
            n = TAIL if t == NFULL else GROUP
            return agg_out.at[c, pl.ds(rbase + t * GROUP, n)]

        def buf_chunk(p, t):
            n = TAIL if t == NFULL else GROUP
            return bufs[p].at[pl.ds(0, n)]

        for t in range(NFULL + 1):
            p = t % 2
            if t >= 2:
                pltpu.make_async_copy(buf_chunk(p, t - 2), out_chunk(t - 2),
                                      ssems[p]).wait()
            pltpu.async_copy(acc_chunk(t), buf_chunk(p, t), gsems[p]).wait()
            pltpu.async_copy(buf_chunk(p, t), out_chunk(t), ssems[p])
        for t in (NFULL - 1, NFULL):
            p = t % 2
            pltpu.make_async_copy(buf_chunk(p, t), out_chunk(t),
                                  ssems[p]).wait()

    return body(src2d, dst2d, x, zrow)


def _tc_body(p_ref, degt_ref, x_ref, wl_ref, wr_ref, bl_ref, g_ref, b_ref,
             o_ref):
    agg = p_ref[0] + p_ref[1]
    deg = jnp.sum(degt_ref[...], axis=1, keepdims=True)
    inv = 1.0 / jnp.maximum(deg, 1.0)
    agg = agg * inv
    x = x_ref[...]
    f = (jnp.dot(agg, wl_ref[...], preferred_element_type=jnp.float32)
         + jnp.dot(x, wr_ref[...], preferred_element_type=jnp.float32)
         + bl_ref[...])
    f = 0.5 * f * (1.0 + lax.erf(f * 0.7071067811865476))
    mu = jnp.mean(f, axis=-1, keepdims=True)
    var = jnp.mean((f - mu) ** 2, axis=-1, keepdims=True)
    f = (f - mu) * lax.rsqrt(var + EPS) * g_ref[...] + b_ref[...]
    o_ref[...] = f + x


def _tc_finish(agg_p, degt, x, W_l, W_r, b_l, gamma, beta):
    R = 1000
    grid = (N_NODES // R,)
    return pl.pallas_call(
        _tc_body,
        grid=grid,
        in_specs=[
            pl.BlockSpec((NC, R, D), lambda i: (0, i, 0)),
            pl.BlockSpec((R, NW), lambda i: (i, 0)),
            pl.BlockSpec((R, D), lambda i: (i, 0)),
            pl.BlockSpec((D, D), lambda i: (0, 0)),
            pl.BlockSpec((D, D), lambda i: (0, 0)),
            pl.BlockSpec((1, D), lambda i: (0, 0)),
            pl.BlockSpec((1, D), lambda i: (0, 0)),
            pl.BlockSpec((1, D), lambda i: (0, 0)),
        ],
        out_specs=pl.BlockSpec((R, D), lambda i: (i, 0)),
        out_shape=jax.ShapeDtypeStruct((N_NODES, D), jnp.float32),
    )(agg_p, degt, x, W_l, W_r, b_l.reshape(1, D), gamma.reshape(1, D),
      beta.reshape(1, D))


def kernel(x, edge_index, batch, W_l, b_l, W_r, gamma, beta):
    src = edge_index[0].astype(jnp.int32)
    dst = edge_index[1].astype(jnp.int32)
    npad = E_PAD - N_EDGES
    src2d = jnp.concatenate(
        [src, jnp.zeros((npad,), jnp.int32)]).reshape(N_GROUPS, GROUP)
    dst2d = jnp.concatenate(
        [dst, jnp.full((npad,), N_NODES, jnp.int32)]).reshape(N_GROUPS, GROUP)
    zrow = jnp.zeros((GROUP, D), jnp.float32)
    agg_p, hist = _sc_aggregate(src2d, dst2d, x, zrow)
    degt = hist.reshape(NW, ROWS_PAD).T
    return _tc_finish(agg_p, degt, x, W_l, W_r, b_l, gamma, beta)


# speedup vs baseline: 1.1623x; 1.0226x over previous
import functools

import jax
import jax.numpy as jnp
from jax import lax
from jax.experimental import pallas as pl
from jax.experimental.pallas import tpu as pltpu
from jax.experimental.pallas import tpu_sc as plsc

N_NODES = 10000
N_EDGES = 320000
D = 128
EPS = 1e-5

NC = 2
NS = 16
NW = NC * NS
LANES = 16
GROUP = 64
E_PAD = 327680
GP_W = (E_PAD // NW) // GROUP
N_GROUPS = E_PAD // GROUP
ROWS_PAD = 10112
RPT = ROWS_PAD // NS
NFULL = RPT // GROUP
TAIL = RPT - NFULL * GROUP


def _sc_aggregate(src2d, dst2d, x, zrow):
    mesh = plsc.VectorSubcoreMesh(core_axis_name="c", subcore_axis_name="s",
                                  num_cores=NC)

    @functools.partial(
        pl.kernel,
        mesh=mesh,
        compiler_params=pltpu.CompilerParams(needs_layout_passes=False),
        out_type=[
            jax.ShapeDtypeStruct((NC, ROWS_PAD, D), jnp.float32),
            jax.ShapeDtypeStruct((NC, NS, ROWS_PAD), jnp.float32),
        ],
        scratch_types=[
            pltpu.VMEM((2, 8, GROUP), jnp.int32),
            pltpu.VMEM((2, 8, GROUP), jnp.int32),
            pltpu.VMEM((GROUP, D), jnp.float32),
            pltpu.VMEM((GROUP, D), jnp.float32),
            pltpu.VMEM((ROWS_PAD,), jnp.float32),
            pltpu.VMEM_SHARED((ROWS_PAD, D), jnp.float32),
            pltpu.SemaphoreType.DMA,
            pltpu.SemaphoreType.DMA,
            pltpu.SemaphoreType.DMA,
            pltpu.SemaphoreType.DMA,
            pltpu.SemaphoreType.DMA,
            pltpu.SemaphoreType.DMA,
        ],
    )
    def body(src_hbm, dst_hbm, x_hbm, zrow_hbm,
             agg_out, hist_out,
             srcv, dstv, rows0, rows1, hist, acc_sh,
             gsem0, gsem1, ssem0, ssem1, isem0, isem1):
        c = lax.axis_index("c")
        s = lax.axis_index("s")
        wid = s * NC + c
        rbase = s * RPT
        gbase = wid * GP_W
        bufs = (rows0, rows1)
        gsems = (gsem0, gsem1)
        ssems = (ssem0, ssem1)
        isems = (isem0, isem1)

        def idx_start(t, q):
            pltpu.async_copy(src_hbm.at[pl.ds(gbase + t * 8, 8)], srcv.at[q],
                             isems[q])
            pltpu.async_copy(dst_hbm.at[pl.ds(gbase + t * 8, 8)], dstv.at[q],
                             isems[q])

        def idx_wait(t, q):
            pltpu.make_async_copy(src_hbm.at[pl.ds(gbase + t * 8, 8)],
                                  srcv.at[q], isems[q]).wait()
            pltpu.make_async_copy(dst_hbm.at[pl.ds(gbase + t * 8, 8)],
                                  dstv.at[q], isems[q]).wait()

        def acc_chunk(t):
            n = TAIL if t == NFULL else GROUP
            return acc_sh.at[pl.ds(rbase + t * GROUP, n)]

        zvec = jnp.zeros((LANES,), jnp.float32)

        def zh(i, carry):
            hist[pl.ds(i * LANES, LANES)] = zvec
            return carry

        lax.fori_loop(0, ROWS_PAD // LANES, zh, 0)

        pltpu.sync_copy(zrow_hbm, rows0)
        for t in range(NFULL):
            pltpu.async_copy(rows0, acc_chunk(t), ssem0)
        pltpu.async_copy(rows0.at[pl.ds(0, TAIL)], acc_chunk(NFULL), ssem1)
        for t in range(NFULL):
            pltpu.make_async_copy(rows0, acc_chunk(t), ssem0).wait()
        pltpu.make_async_copy(rows0.at[pl.ds(0, TAIL)], acc_chunk(NFULL),
                              ssem1).wait()
        plsc.subcore_barrier()

        idx_start(0, 0)

        def outer(t, carry):
            for q in (0, 1):
                tc = 2 * t + q
                idx_wait(tc, q)
                pltpu.async_copy(x_hbm.at[srcv.at[q, 0]], rows0, gsem0)
                pltpu.async_copy(x_hbm.at[srcv.at[q, 1]], rows1, gsem1)

                @pl.when(tc + 1 < GP_W // 8)
                def _():
                    idx_start(tc + 1, 1 - q)

                for j in range(8):
                    for k in range(GROUP // LANES):
                        d16 = dstv[q, j, pl.ds(k * LANES, LANES)]
                        cnt, last = plsc.scan_count(d16)
                        plsc.addupdate_scatter(hist, [d16],
                                               cnt.astype(jnp.float32),
                                               mask=last)
                for j in range(8):
                    p = j % 2
                    pltpu.make_async_copy(x_hbm.at[srcv.at[q, j]], bufs[p],
                                          gsems[p]).wait()
                    pltpu.async_copy(bufs[p], acc_sh.at[dstv.at[q, j]],
                                     ssems[p], add=True)
                    if j + 2 < 8:
                        pltpu.make_async_copy(bufs[p], acc_sh.at[dstv.at[q, j]],
                                              ssems[p]).wait()
                        pltpu.async_copy(x_hbm.at[srcv.at[q, j + 2]], bufs[p],
                                         gsems[p])
                for j in (6, 7):
                    p = j % 2
                    pltpu.make_async_copy(bufs[p], acc_sh.at[dstv.at[q, j]],
                                          ssems[p]).wait()
            return carry

        lax.fori_loop(0, GP_W // 16, outer, 0)
        pltpu.sync_copy(hist, hist_out.at[c, s])
        plsc.subcore_barrier()

        def out_chunk(t):
            n = TAIL if t == NFULL else GROUP
            return agg_out.at[c, pl.ds(rbase + t * GROUP, n)]

        def buf_chunk(p, t):
            n = TAIL if t == NFULL else GROUP
            return bufs[p].at[pl.ds(0, n)]

        for t in range(NFULL + 1):
            p = t % 2
            if t >= 2:
                pltpu.make_async_copy(buf_chunk(p, t - 2), out_chunk(t - 2),
                                      ssems[p]).wait()
            pltpu.async_copy(acc_chunk(t), buf_chunk(p, t), gsems[p]).wait()
            pltpu.async_copy(buf_chunk(p, t), out_chunk(t), ssems[p])
        for t in (NFULL - 1, NFULL):
            p = t % 2
            pltpu.make_async_copy(buf_chunk(p, t), out_chunk(t),
                                  ssems[p]).wait()

    return body(src2d, dst2d, x, zrow)


def _tc_body(p_ref, degt_ref, x_ref, wl_ref, wr_ref, bl_ref, g_ref, b_ref,
             o_ref):
    agg = p_ref[0] + p_ref[1]
    deg = jnp.sum(degt_ref[...], axis=1, keepdims=True)
    inv = 1.0 / jnp.maximum(deg, 1.0)
    agg = agg * inv
    x = x_ref[...]
    f = (jnp.dot(agg, wl_ref[...], preferred_element_type=jnp.float32)
         + jnp.dot(x, wr_ref[...], preferred_element_type=jnp.float32)
         + bl_ref[...])
    f = 0.5 * f * (1.0 + lax.erf(f * 0.7071067811865476))
    mu = jnp.mean(f, axis=-1, keepdims=True)
    var = jnp.mean((f - mu) ** 2, axis=-1, keepdims=True)
    f = (f - mu) * lax.rsqrt(var + EPS) * g_ref[...] + b_ref[...]
    o_ref[...] = f + x


def _tc_finish(agg_p, degt, x, W_l, W_r, b_l, gamma, beta):
    R = 1000
    grid = (N_NODES // R,)
    return pl.pallas_call(
        _tc_body,
        grid=grid,
        in_specs=[
            pl.BlockSpec((NC, R, D), lambda i: (0, i, 0)),
            pl.BlockSpec((R, NW), lambda i: (i, 0)),
            pl.BlockSpec((R, D), lambda i: (i, 0)),
            pl.BlockSpec((D, D), lambda i: (0, 0)),
            pl.BlockSpec((D, D), lambda i: (0, 0)),
            pl.BlockSpec((1, D), lambda i: (0, 0)),
            pl.BlockSpec((1, D), lambda i: (0, 0)),
            pl.BlockSpec((1, D), lambda i: (0, 0)),
        ],
        out_specs=pl.BlockSpec((R, D), lambda i: (i, 0)),
        out_shape=jax.ShapeDtypeStruct((N_NODES, D), jnp.float32),
    )(agg_p, degt, x, W_l, W_r, b_l.reshape(1, D), gamma.reshape(1, D),
      beta.reshape(1, D))


def kernel(x, edge_index, batch, W_l, b_l, W_r, gamma, beta):
    src = edge_index[0].astype(jnp.int32)
    dst = edge_index[1].astype(jnp.int32)
    npad = E_PAD - N_EDGES
    src2d = jnp.concatenate(
        [src, jnp.zeros((npad,), jnp.int32)]).reshape(N_GROUPS, GROUP)
    dst2d = jnp.concatenate(
        [dst, jnp.full((npad,), N_NODES, jnp.int32)]).reshape(N_GROUPS, GROUP)
    zrow = jnp.zeros((GROUP, D), jnp.float32)
    agg_p, hist = _sc_aggregate(src2d, dst2d, x, zrow)
    degt = hist.reshape(NW, ROWS_PAD).T
    return _tc_finish(agg_p, degt, x, W_l, W_r, b_l, gamma, beta)
